# Initial kernel scaffold; baseline (speedup 1.0000x reference)
import jax
import jax.numpy as jnp
from jax import lax
from jax.experimental import pallas as pl
from jax.experimental.pallas import tpu as pltpu
from jax.experimental.pallas import tpu_sc as plsc

D = 64
LANES = 16
NC, NS = 2, 16
NW = NC * NS
SUB = 128


def _make_kernel(n_rows, chunk, interpret=False):
    per_w = n_rows // NW
    n_chunks = per_w // chunk
    n_sub = chunk // SUB
    mesh = plsc.VectorSubcoreMesh(core_axis_name="c", subcore_axis_name="s")

    def body(seq_hbm, t_hbm, r_hbm, tok_hbm, tim_hbm, rat_hbm,
             x_out, tim_out,
             seq_i, t_i, r_i, acc_v, tim_v, rat_v, sem):
        wid = lax.axis_index("s") * NC + lax.axis_index("c")
        sub_w = wid * (per_w // SUB)

        def chunk_body(c, carry):
            sub0 = sub_w + c * n_sub
            base = wid * per_w + c * chunk
            pltpu.sync_copy(seq_hbm.at[pl.ds(sub0, n_sub)], seq_i)
            pltpu.sync_copy(t_hbm.at[pl.ds(sub0, n_sub)], t_i)
            pltpu.sync_copy(r_hbm.at[pl.ds(sub0, n_sub)], r_i)
            cps = []
            for j in range(n_sub):
                dst = pl.ds(j * SUB, SUB)
                cps.append(pltpu.async_copy(
                    tok_hbm.at[seq_i.at[j]], acc_v.at[dst], sem))
                cps.append(pltpu.async_copy(
                    tim_hbm.at[t_i.at[j]], tim_v.at[dst], sem))
                cps.append(pltpu.async_copy(
                    rat_hbm.at[r_i.at[j]], rat_v.at[dst], sem))
            for cp in cps:
                cp.wait()

            def add_row(i, inner):
                for g in range(D // LANES):
                    sl = pl.ds(g * LANES, LANES)
                    acc_v[i, sl] = acc_v[i, sl] + tim_v[i, sl] + rat_v[i, sl]
                return inner
            lax.fori_loop(0, chunk, add_row, 0)

            pltpu.sync_copy(acc_v, x_out.at[pl.ds(base, chunk)])
            pltpu.sync_copy(tim_v, tim_out.at[pl.ds(base, chunk)])
            return carry
        lax.fori_loop(0, n_chunks, chunk_body, 0)

    return pl.kernel(
        body,
        out_type=(jax.ShapeDtypeStruct((n_rows, D), jnp.float32),
                  jax.ShapeDtypeStruct((n_rows, D), jnp.float32)),
        mesh=mesh,
        scratch_types=[
            pltpu.VMEM((n_sub, SUB), jnp.int32),
            pltpu.VMEM((n_sub, SUB), jnp.int32),
            pltpu.VMEM((n_sub, SUB), jnp.int32),
            pltpu.VMEM((chunk, D), jnp.float32),
            pltpu.VMEM((chunk, D), jnp.float32),
            pltpu.VMEM((chunk, D), jnp.float32),
            pltpu.SemaphoreType.DMA,
        ],
        interpret=interpret,
    )


def kernel(sequence, r, t, tok_table, rat_table, tim_table):
    B_, L_ = sequence.shape
    n = B_ * L_
    seq2 = sequence.reshape(n // SUB, SUB).astype(jnp.int32)
    t2 = t.reshape(n // SUB, SUB).astype(jnp.int32)
    r2 = r.reshape(n // SUB, SUB).astype(jnp.int32)
    k = _make_kernel(n, 512)
    x, tim = k(seq2, t2, r2, tok_table, tim_table, rat_table)
    return x.reshape(B_, L_, D), tim.reshape(B_, L_, D)


# speedup vs baseline: 1.1321x; 1.1321x over previous
import jax
import jax.numpy as jnp
from jax import lax
from jax.experimental import pallas as pl
from jax.experimental.pallas import tpu as pltpu
from jax.experimental.pallas import tpu_sc as plsc

D = 64
LANES = 16
NC, NS = 2, 16
NW = NC * NS
SUB = 128


def _make_kernel(n_rows, chunk, interpret=False):
    per_w = n_rows // NW
    n_chunks = per_w // chunk
    n_sub = chunk // SUB
    mesh = plsc.VectorSubcoreMesh(core_axis_name="c", subcore_axis_name="s",
                                  num_cores=NC, num_subcores=NS)

    def body(seq_hbm, t_hbm, r_hbm, tok_hbm, tim_hbm, rat_hbm,
             x_out, tim_out,
             seq_i, t_i, r_i, acc_v, tim_v, rat_v, sem):
        wid = lax.axis_index("s") * NC + lax.axis_index("c")
        sub_w = wid * (per_w // SUB)

        def chunk_body(c, carry):
            sub0 = sub_w + c * n_sub
            base = wid * per_w + c * chunk
            pltpu.sync_copy(seq_hbm.at[pl.ds(sub0, n_sub)], seq_i)
            pltpu.sync_copy(t_hbm.at[pl.ds(sub0, n_sub)], t_i)
            pltpu.sync_copy(r_hbm.at[pl.ds(sub0, n_sub)], r_i)
            cps = []
            for j in range(n_sub):
                dst = pl.ds(j * SUB, SUB)
                cps.append(pltpu.async_copy(
                    tok_hbm.at[seq_i.at[j]], acc_v.at[dst], sem))
                cps.append(pltpu.async_copy(
                    tim_hbm.at[t_i.at[j]], tim_v.at[dst], sem))
                cps.append(pltpu.async_copy(
                    rat_hbm.at[r_i.at[j]], rat_v.at[dst], sem))
            for cp in cps:
                cp.wait()

            def add_row(i, inner):
                for g in range(D // LANES):
                    sl = pl.ds(g * LANES, LANES)
                    acc_v[i, sl] = acc_v[i, sl] + tim_v[i, sl] + rat_v[i, sl]
                return inner
            lax.fori_loop(0, chunk, add_row, 0)

            pltpu.sync_copy(acc_v, x_out.at[pl.ds(base, chunk)])
            pltpu.sync_copy(tim_v, tim_out.at[pl.ds(base, chunk)])
            return carry
        lax.fori_loop(0, n_chunks, chunk_body, 0)

    return pl.kernel(
        body,
        out_type=(jax.ShapeDtypeStruct((n_rows, D), jnp.float32),
                  jax.ShapeDtypeStruct((n_rows, D), jnp.float32)),
        mesh=mesh,
        scratch_types=[
            pltpu.VMEM((n_sub, SUB), jnp.int32),
            pltpu.VMEM((n_sub, SUB), jnp.int32),
            pltpu.VMEM((n_sub, SUB), jnp.int32),
            pltpu.VMEM((chunk, D), jnp.float32),
            pltpu.VMEM((chunk, D), jnp.float32),
            pltpu.VMEM((chunk, D), jnp.float32),
            pltpu.SemaphoreType.DMA,
        ],
        compiler_params=pltpu.CompilerParams(use_tc_tiling_on_sc=False),
        interpret=interpret,
    )


def kernel(sequence, r, t, tok_table, rat_table, tim_table):
    B_, L_ = sequence.shape
    n = B_ * L_
    seq2 = sequence.reshape(n // SUB, SUB).astype(jnp.int32)
    t2 = t.reshape(n // SUB, SUB).astype(jnp.int32)
    r2 = r.reshape(n // SUB, SUB).astype(jnp.int32)
    k = _make_kernel(n, 512)
    x, tim = k(seq2, t2, r2, tok_table, tim_table, rat_table)
    return x.reshape(B_, L_, D), tim.reshape(B_, L_, D)


# speedup vs baseline: 3.2171x; 2.8417x over previous
import jax
import jax.numpy as jnp
from jax import lax
from jax.experimental import pallas as pl
from jax.experimental.pallas import tpu as pltpu
from jax.experimental.pallas import tpu_sc as plsc

D = 64
LANES = 16
NC, NS = 2, 16
NW = NC * NS
SUB = 128
CHUNK = 256
NSUB = CHUNK // SUB


def _make_kernel(n_rows, t_rows, r_rows):
    per_w = n_rows // NW
    n_chunks = per_w // CHUNK
    assert n_chunks % 2 == 0 and per_w % CHUNK == 0

    mesh = plsc.VectorSubcoreMesh(core_axis_name="c", subcore_axis_name="s",
                                  num_cores=NC, num_subcores=NS)

    def body(seq_hbm, t_hbm, r_hbm, tok_hbm, tim_hbm, rat_hbm,
             x_out, tim_out,
             timtbl, rattbl,
             seq_i, t_i, r_i, acc, timb, gsem, isem, wsem):
        wid = lax.axis_index("s") * NC + lax.axis_index("c")
        sub_w = wid * (per_w // SUB)
        row_w = wid * per_w

        pltpu.sync_copy(tim_hbm, timtbl)
        pltpu.sync_copy(rat_hbm, rattbl)

        def fire_idx(c, b):
            sub0 = sub_w + c * NSUB
            base = row_w + c * CHUNK
            pltpu.async_copy(seq_hbm.at[pl.ds(sub0, NSUB)], seq_i.at[b],
                             isem.at[b])
            pltpu.async_copy(t_hbm.at[pl.ds(base, CHUNK)], t_i.at[b],
                             isem.at[b])
            pltpu.async_copy(r_hbm.at[pl.ds(base, CHUNK)], r_i.at[b],
                             isem.at[b])

        def wait_idx(b):
            pltpu.make_async_copy(t_hbm.at[pl.ds(0, CHUNK)], t_i.at[b],
                                  isem.at[b]).wait()
            pltpu.make_async_copy(r_hbm.at[pl.ds(0, CHUNK)], r_i.at[b],
                                  isem.at[b]).wait()
            pltpu.make_async_copy(seq_hbm.at[pl.ds(0, NSUB)], seq_i.at[b],
                                  isem.at[b]).wait()

        def fire_gather(b):
            for j in range(NSUB):
                pltpu.async_copy(tok_hbm.at[seq_i.at[b, j]],
                                 acc.at[b, pl.ds(j * SUB, SUB)], gsem.at[b])

        def wait_gather(b):
            for j in range(NSUB):
                pltpu.make_async_copy(tok_hbm.at[seq_i.at[b, j]],
                                      acc.at[b, pl.ds(j * SUB, SUB)],
                                      gsem.at[b]).wait()

        def fire_wb(c, b):
            base = row_w + c * CHUNK
            pltpu.async_copy(acc.at[b], x_out.at[pl.ds(base, CHUNK)],
                             wsem.at[b])
            pltpu.async_copy(timb.at[b], tim_out.at[pl.ds(base, CHUNK)],
                             wsem.at[b])

        def wait_wb(b):
            pltpu.make_async_copy(acc.at[b], x_out.at[pl.ds(0, CHUNK)],
                                  wsem.at[b]).wait()
            pltpu.make_async_copy(timb.at[b], tim_out.at[pl.ds(0, CHUNK)],
                                  wsem.at[b]).wait()

        def compute(b):
            @plsc.parallel_loop(0, CHUNK, step=LANES)
            def _(i0):
                tv16 = t_i[b, pl.ds(i0, LANES)]
                rv16 = r_i[b, pl.ds(i0, LANES)]
                for m in range(LANES):
                    tt = tv16[m]
                    rr = rv16[m]
                    i = i0 + m
                    for g in range(D // LANES):
                        sl = pl.ds(g * LANES, LANES)
                        tv = timtbl[tt, sl]
                        timb[b, i, sl] = tv
                        acc[b, i, sl] = acc[b, i, sl] + tv + rattbl[rr, sl]

        fire_idx(0, 0)
        fire_idx(1, 1)
        wait_idx(0)
        fire_gather(0)

        def pair_body(g, carry):
            for b in range(2):
                c = g * 2 + b
                nb = 1 - b

                @pl.when(c >= 1)
                def _():
                    wait_wb(nb)

                @pl.when(c + 1 < n_chunks)
                def _():
                    wait_idx(nb)
                    fire_gather(nb)

                wait_gather(b)
                compute(b)

                @pl.when(c + 2 < n_chunks)
                def _():
                    fire_idx(c + 2, b)

                fire_wb(c, b)
            return carry
        lax.fori_loop(0, n_chunks // 2, pair_body, 0)
        wait_wb((n_chunks - 1) % 2)

    return pl.kernel(
        body,
        out_type=(jax.ShapeDtypeStruct((n_rows, D), jnp.float32),
                  jax.ShapeDtypeStruct((n_rows, D), jnp.float32)),
        mesh=mesh,
        scratch_types=[
            pltpu.VMEM((t_rows, D), jnp.float32),
            pltpu.VMEM((r_rows, D), jnp.float32),
            pltpu.VMEM((2, NSUB, SUB), jnp.int32),
            pltpu.VMEM((2, CHUNK), jnp.int32),
            pltpu.VMEM((2, CHUNK), jnp.int32),
            pltpu.VMEM((2, CHUNK, D), jnp.float32),
            pltpu.VMEM((2, CHUNK, D), jnp.float32),
            pltpu.SemaphoreType.DMA((2,)),
            pltpu.SemaphoreType.DMA((2,)),
            pltpu.SemaphoreType.DMA((2,)),
        ],
        compiler_params=pltpu.CompilerParams(use_tc_tiling_on_sc=False),
    )


def kernel(sequence, r, t, tok_table, rat_table, tim_table):
    B_, L_ = sequence.shape
    n = B_ * L_
    seq2 = sequence.reshape(n // SUB, SUB).astype(jnp.int32)
    t1 = t.reshape(n).astype(jnp.int32)
    r1 = r.reshape(n).astype(jnp.int32)
    k = _make_kernel(n, tim_table.shape[0], rat_table.shape[0])
    x, tim = k(seq2, t1, r1, tok_table, tim_table, rat_table)
    return x.reshape(B_, L_, D), tim.reshape(B_, L_, D)
